# baseline (speedup 1.0000x reference)
import functools

import jax
import jax.numpy as jnp
from jax import lax
from jax.experimental import pallas as pl
from jax.experimental.pallas import tpu as pltpu
from jax.experimental.pallas import tpu_sc as plsc

N_NODES = 10000
N_PAD = 10240
N_REL = 4
E_TOT = 160000
LANES = 16
NC = 2
NSUB = 16
NW = NC * NSUB
E_TILES = E_TOT // 128
CT_PER_W = E_TILES // NW
CT_REM = E_TILES - CT_PER_W * NW
EPT_MAX = (CT_PER_W + CT_REM) * 128
GROUPS = EPT_MAX // LANES
HIST_ROWS = 2 * N_REL * N_PAD // 128


@functools.cache
def _get_sc_hist():
    mesh = plsc.VectorSubcoreMesh(core_axis_name="c", subcore_axis_name="s")
    return functools.partial(
        pl.kernel,
        mesh=mesh,
        out_type=jax.ShapeDtypeStruct((NC * HIST_ROWS, 128), jnp.float32),
        compiler_params=pltpu.CompilerParams(needs_layout_passes=False),
        scratch_types=[
            pltpu.VMEM((HIST_ROWS, 128), jnp.float32),
            pltpu.VMEM((2, EPT_MAX), jnp.int32),
            pltpu.VMEM((EPT_MAX,), jnp.int32),
            pltpu.VMEM((HIST_ROWS,), jnp.int32),
            pltpu.VMEM_SHARED((HIST_ROWS, 128), jnp.float32),
        ],
    )(_sc_hist)


def _sc_hist(ei_hbm, typ_hbm, out_hbm, hist, s_ei, s_typ, rowidx, shared):
    cid = lax.axis_index("c")
    sid = lax.axis_index("s")
    wid = sid * NC + cid
    base = wid * (CT_PER_W * 128)
    count = jnp.where(wid == NW - 1, EPT_MAX, CT_PER_W * 128)

    zeros16 = jnp.zeros((LANES,), jnp.float32)

    @plsc.parallel_loop(0, HIST_ROWS, unroll=2)
    def _zero(i):
        for j in range(8):
            hist[i, pl.ds(j * LANES, LANES)] = zeros16

    lane = lax.iota(jnp.int32, LANES)

    rows_per_tile = HIST_ROWS // NSUB
    pltpu.sync_copy(hist.at[pl.ds(sid * rows_per_tile, rows_per_tile)],
                    shared.at[pl.ds(sid * rows_per_tile, rows_per_tile)])

    @plsc.parallel_loop(0, HIST_ROWS // LANES, unroll=2)
    def _iota(g):
        rowidx[pl.ds(g * LANES, LANES)] = g * LANES + lane

    pltpu.sync_copy(ei_hbm.at[:, pl.ds(base, EPT_MAX)], s_ei)
    pltpu.sync_copy(typ_hbm.at[pl.ds(base, EPT_MAX)], s_typ)

    ones16 = jnp.full((LANES,), 1.0, jnp.float32)

    @plsc.parallel_loop(0, GROUPS, unroll=4)
    def _scatter(i):
        off = i * LANES
        msk = lane < jnp.minimum(count - off, LANES)
        sr = s_ei[0, pl.ds(off, LANES)]
        ds_ = s_ei[1, pl.ds(off, LANES)]
        ty = s_typ[pl.ds(off, LANES)]
        row1 = ((ds_ >> 7) << 3) + ty
        plsc.addupdate_scatter(hist, [row1, ds_ & 127], ones16, mask=msk)
        row2 = ((sr >> 7) << 3) + (N_REL + ty)
        plsc.addupdate_scatter(hist, [row2, sr & 127], ones16,
                               mask=msk & (ds_ == 0))

    plsc.subcore_barrier()
    pltpu.sync_copy(hist, shared.at[rowidx], add=True)
    plsc.subcore_barrier()
    pltpu.sync_copy(
        shared.at[pl.ds(sid * rows_per_tile, rows_per_tile)],
        out_hbm.at[pl.ds(cid * HIST_ROWS + sid * rows_per_tile,
                         rows_per_tile)])


XCH = 1280
NSTEPS_X = N_PAD // XCH
GCH = 2048
SCH = 1024
NSTEPS = NSTEPS_X + 8 + 1


def _tc_body(h_ref, x_ref, cw_ref, w0_ref, gate_ref, mem_ref,
             wgt_ref, bg_ref, wst_ref, bs_ref, pg_ref, ps_ref,
             s_acc, x1_s):
    i = pl.program_id(0)

    @pl.when(i == 0)
    def _init():
        s_acc[...] = jnp.zeros((N_REL, 128), jnp.float32)

    @pl.when(i < NSTEPS_X)
    def _accum():
        hb = h_ref[...]
        hs = hb[0] + hb[1]
        h3 = hs.reshape(10, 2 * N_REL, 128)
        dinv3 = lax.rsqrt(h3[:, 0:N_REL, :] + 1.0)
        c3 = h3[:, N_REL:2 * N_REL, :] * dinv3
        xb = x_ref[...]
        valid = jnp.minimum(N_NODES - i * XCH, XCH)
        rows = lax.broadcasted_iota(jnp.int32, (XCH, 128), 0)
        xb = jnp.where(rows < valid, xb, 0.0)
        x3 = xb.reshape(10, 128, 128)
        sb = lax.dot_general(c3, x3, (((2,), (1,)), ((0,), (0,))),
                             preferred_element_type=jnp.float32)
        s_acc[...] = s_acc[...] + jnp.sum(sb, axis=0)

    @pl.when(i == NSTEPS_X)
    def _finalize_x1():
        hb = h_ref[...]
        hs = hb[0] + hb[1]
        dinv0 = lax.rsqrt(hs[0:N_REL, 0:1] + 1.0)
        x0 = x_ref[0:1, :]
        y = dinv0 * s_acc[...] + (dinv0 * dinv0) * x0
        comp = jnp.zeros((1, 128), jnp.float32)
        for r in range(N_REL):
            comp = comp + lax.dot_general(
                y[r:r + 1, :], cw_ref[r],
                (((1,), (1,)), ((), ())),
                preferred_element_type=jnp.float32)
        prev = jnp.dot(x0, w0_ref[...], preferred_element_type=jnp.float32)
        g = gate_ref[...]
        rg = g * (comp + prev) + (1.0 - g) * mem_ref[...]
        x1_s[...] = jnp.maximum(rg, 0.0)

    for k in range(8):
        @pl.when(i == NSTEPS_X + k)
        def _gchunk(k=k):
            logits = lax.dot_general(
                x1_s[...], wgt_ref[...], (((1,), (1,)), ((), ())),
                preferred_element_type=jnp.float32) + bg_ref[...]
            pg_ref[:, k * GCH:(k + 1) * GCH] = logits

    for k in range(4):
        @pl.when(i == NSTEPS_X + k)
        def _schunk(k=k):
            logits = lax.dot_general(
                x1_s[...], wst_ref[...], (((1,), (1,)), ((), ())),
                preferred_element_type=jnp.float32) + bs_ref[...]
            ps_ref[:, k * SCH:(k + 1) * SCH] = logits

    @pl.when(i == NSTEPS - 1)
    def _logsoftmax():
        for o_ref in (pg_ref, ps_ref):
            lg = o_ref[...]
            mx = jnp.max(lg, axis=1, keepdims=True)
            lse = jnp.log(jnp.sum(jnp.exp(lg - mx), axis=1,
                                  keepdims=True)) + mx
            o_ref[...] = lg - lse


def kernel(x, edge_index, edge_type, conv_W, W0, update_gate,
           lin2g_W, lin2g_b, lin2s_W, lin2s_b, memory_prev):
    hist = _get_sc_hist()(edge_index, edge_type)

    ng = lin2g_W.shape[0]
    ns = lin2s_W.shape[0]
    first8 = lambda i: jnp.where(i < NSTEPS_X, i, 0)
    gidx = lambda i: jnp.clip(i - NSTEPS_X, 0, ng // GCH - 1)
    sidx = lambda i: jnp.clip(i - NSTEPS_X, 0, ns // SCH - 1)
    pg, ps = pl.pallas_call(
        _tc_body,
        grid=(NSTEPS,),
        in_specs=[
            pl.BlockSpec((NC, 80, 128), lambda i: (0, first8(i), 0)),
            pl.BlockSpec((XCH, 128), lambda i: (first8(i), 0)),
            pl.BlockSpec((N_REL, 128, 128), lambda i: (0, 0, 0)),
            pl.BlockSpec((128, 128), lambda i: (0, 0)),
            pl.BlockSpec((1, 1), lambda i: (0, 0)),
            pl.BlockSpec((1, 128), lambda i: (0, 0)),
            pl.BlockSpec((GCH, 128), lambda i: (gidx(i), 0)),
            pl.BlockSpec((1, GCH), lambda i: (0, gidx(i))),
            pl.BlockSpec((SCH, 128), lambda i: (sidx(i), 0)),
            pl.BlockSpec((1, SCH), lambda i: (0, sidx(i))),
        ],
        out_specs=(
            pl.BlockSpec((1, ng), lambda i: (0, 0)),
            pl.BlockSpec((1, ns), lambda i: (0, 0)),
        ),
        out_shape=(
            jax.ShapeDtypeStruct((1, ng), jnp.float32),
            jax.ShapeDtypeStruct((1, ns), jnp.float32),
        ),
        scratch_shapes=[
            pltpu.VMEM((N_REL, 128), jnp.float32),
            pltpu.VMEM((1, 128), jnp.float32),
        ],
    )(hist.reshape(NC, HIST_ROWS, 128), x, conv_W, W0,
      update_gate.reshape(1, 1), memory_prev.reshape(1, -1),
      lin2g_W, lin2g_b.reshape(1, -1),
      lin2s_W, lin2s_b.reshape(1, -1))
    return (pg, ps)


# speedup vs baseline: 1.1455x; 1.1455x over previous
import functools

import jax
import jax.numpy as jnp
from jax import lax
from jax.experimental import pallas as pl
from jax.experimental.pallas import tpu as pltpu
from jax.experimental.pallas import tpu_sc as plsc

N_NODES = 10000
N_PAD = 10240
N_REL = 4
E_TOT = 160000
LANES = 16
NC = 2
NSUB = 16
NW = NC * NSUB
E_TILES = E_TOT // 128
CT_PER_W = E_TILES // NW
CT_REM = E_TILES - CT_PER_W * NW
EPT_MAX = (CT_PER_W + CT_REM) * 128
GROUPS = EPT_MAX // LANES
HIST_ROWS = 2 * N_REL * N_PAD // 128


@functools.cache
def _get_sc_hist():
    mesh = plsc.VectorSubcoreMesh(core_axis_name="c", subcore_axis_name="s")
    return functools.partial(
        pl.kernel,
        mesh=mesh,
        out_type=jax.ShapeDtypeStruct((NC * HIST_ROWS, 128), jnp.float32),
        compiler_params=pltpu.CompilerParams(needs_layout_passes=False),
        scratch_types=[
            pltpu.VMEM((HIST_ROWS, 128), jnp.float32),
            pltpu.VMEM((2, EPT_MAX), jnp.int32),
            pltpu.VMEM((EPT_MAX,), jnp.int32),
            pltpu.VMEM((HIST_ROWS,), jnp.int32),
            pltpu.VMEM_SHARED((HIST_ROWS, 128), jnp.float32),
        ],
    )(_sc_hist)


def _sc_hist(ei_hbm, typ_hbm, out_hbm, hist, s_ei, s_typ, rowidx, shared):
    cid = lax.axis_index("c")
    sid = lax.axis_index("s")
    wid = sid * NC + cid
    base = wid * (CT_PER_W * 128)
    count = jnp.where(wid == NW - 1, EPT_MAX, CT_PER_W * 128)

    zeros16 = jnp.zeros((LANES,), jnp.float32)

    @plsc.parallel_loop(0, HIST_ROWS, unroll=4)
    def _zero(i):
        for j in range(8):
            hist[i, pl.ds(j * LANES, LANES)] = zeros16

    lane = lax.iota(jnp.int32, LANES)

    rows_per_tile = HIST_ROWS // NSUB
    pltpu.sync_copy(hist.at[pl.ds(sid * rows_per_tile, rows_per_tile)],
                    shared.at[pl.ds(sid * rows_per_tile, rows_per_tile)])

    @plsc.parallel_loop(0, HIST_ROWS // LANES, unroll=2)
    def _iota(g):
        rowidx[pl.ds(g * LANES, LANES)] = g * LANES + lane

    pltpu.sync_copy(ei_hbm.at[:, pl.ds(base, EPT_MAX)], s_ei)
    pltpu.sync_copy(typ_hbm.at[pl.ds(base, EPT_MAX)], s_typ)

    ones16 = jnp.full((LANES,), 1.0, jnp.float32)

    @plsc.parallel_loop(0, GROUPS, unroll=8)
    def _scatter(i):
        off = i * LANES
        msk = lane < jnp.minimum(count - off, LANES)
        sr = s_ei[0, pl.ds(off, LANES)]
        ds_ = s_ei[1, pl.ds(off, LANES)]
        ty = s_typ[pl.ds(off, LANES)]
        row1 = ((ds_ >> 7) << 3) + ty
        plsc.addupdate_scatter(hist, [row1, ds_ & 127], ones16, mask=msk)
        row2 = ((sr >> 7) << 3) + (N_REL + ty)
        plsc.addupdate_scatter(hist, [row2, sr & 127], ones16,
                               mask=msk & (ds_ == 0))

    plsc.subcore_barrier()
    pltpu.sync_copy(hist, shared.at[rowidx], add=True)
    plsc.subcore_barrier()
    pltpu.sync_copy(
        shared.at[pl.ds(sid * rows_per_tile, rows_per_tile)],
        out_hbm.at[pl.ds(cid * HIST_ROWS + sid * rows_per_tile,
                         rows_per_tile)])


def _tc_body(h_ref, x_ref, cw_ref, w0_ref, gate_ref, mem_ref,
             wgt_ref, bg_ref, wst_ref, bs_ref, pg_ref, ps_ref):
    h = h_ref[...].reshape(NC, HIST_ROWS, 128)
    hsum = jnp.sum(h, axis=0)
    h3 = hsum.reshape(N_PAD // 128, 2 * N_REL, 128)
    dinv3 = lax.rsqrt(h3[:, 0:N_REL, :] + 1.0)
    c3 = h3[:, N_REL:2 * N_REL, :] * dinv3

    x = x_ref[...]
    xp = jnp.concatenate(
        [x, jnp.zeros((N_PAD - N_NODES, x.shape[1]), jnp.float32)], axis=0)
    x3 = xp.reshape(N_PAD // 128, 128, x.shape[1])
    sb = lax.dot_general(c3, x3, (((2,), (1,)), ((0,), (0,))),
                         preferred_element_type=jnp.float32)
    s = jnp.sum(sb, axis=0)

    dinv0 = dinv3[0, :, 0:1]
    x0 = x[0:1, :]
    y = dinv0 * s + (dinv0 * dinv0) * x0

    comp = jnp.zeros((1, x.shape[1]), jnp.float32)
    for r in range(N_REL):
        comp = comp + lax.dot_general(
            y[r:r + 1, :], cw_ref[r],
            (((1,), (1,)), ((), ())),
            preferred_element_type=jnp.float32)

    prev = jnp.dot(x0, w0_ref[...], preferred_element_type=jnp.float32)
    g = gate_ref[...]
    rg = g * (comp + prev) + (1.0 - g) * mem_ref[...]
    x1 = jnp.maximum(rg, 0.0)

    def head(wt_ref, b_ref, o_ref):
        logits = lax.dot_general(
            x1, wt_ref[...], (((1,), (1,)), ((), ())),
            preferred_element_type=jnp.float32) + b_ref[...]
        mx = jnp.max(logits, axis=1, keepdims=True)
        lse = jnp.log(jnp.sum(jnp.exp(logits - mx), axis=1,
                              keepdims=True)) + mx
        o_ref[...] = logits - lse

    head(wgt_ref, bg_ref, pg_ref)
    head(wst_ref, bs_ref, ps_ref)


def kernel(x, edge_index, edge_type, conv_W, W0, update_gate,
           lin2g_W, lin2g_b, lin2s_W, lin2s_b, memory_prev):
    hist = _get_sc_hist()(edge_index, edge_type)

    ng = lin2g_W.shape[0]
    ns = lin2s_W.shape[0]
    pg, ps = pl.pallas_call(
        _tc_body,
        out_shape=(
            jax.ShapeDtypeStruct((1, ng), jnp.float32),
            jax.ShapeDtypeStruct((1, ns), jnp.float32),
        ),
    )(hist, x, conv_W, W0,
      update_gate.reshape(1, 1), memory_prev.reshape(1, -1),
      lin2g_W, lin2g_b.reshape(1, -1),
      lin2s_W, lin2s_b.reshape(1, -1))
    return (pg, ps)


# speedup vs baseline: 1.1518x; 1.0055x over previous
import functools

import jax
import jax.numpy as jnp
from jax import lax
from jax.experimental import pallas as pl
from jax.experimental.pallas import tpu as pltpu
from jax.experimental.pallas import tpu_sc as plsc

N_NODES = 10000
N_PAD = 10240
N_REL = 4
E_TOT = 160000
LANES = 16
NC = 2
NSUB = 16
NW = NC * NSUB
E_TILES = E_TOT // 128
CT_PER_W = E_TILES // NW
CT_REM = E_TILES - CT_PER_W * NW
EPT_MAX = (CT_PER_W + CT_REM) * 128
GROUPS = EPT_MAX // LANES
HIST_ROWS = 2 * N_REL * N_PAD // 128


@functools.cache
def _get_sc_hist():
    mesh = plsc.VectorSubcoreMesh(core_axis_name="c", subcore_axis_name="s")
    return functools.partial(
        pl.kernel,
        mesh=mesh,
        out_type=jax.ShapeDtypeStruct((NC * HIST_ROWS, 128), jnp.float32),
        compiler_params=pltpu.CompilerParams(needs_layout_passes=False),
        scratch_types=[
            pltpu.VMEM((HIST_ROWS, 128), jnp.float32),
            pltpu.VMEM((2, EPT_MAX), jnp.int32),
            pltpu.VMEM((EPT_MAX,), jnp.int32),
            pltpu.VMEM((HIST_ROWS,), jnp.int32),
            pltpu.VMEM_SHARED((HIST_ROWS, 128), jnp.float32),
        ],
    )(_sc_hist)


def _sc_hist(ei_hbm, typ_hbm, out_hbm, hist, s_ei, s_typ, rowidx, shared):
    cid = lax.axis_index("c")
    sid = lax.axis_index("s")
    wid = sid * NC + cid
    base = wid * (CT_PER_W * 128)
    count = jnp.where(wid == NW - 1, EPT_MAX, CT_PER_W * 128)

    zeros16 = jnp.zeros((LANES,), jnp.float32)

    @plsc.parallel_loop(0, HIST_ROWS, unroll=2)
    def _zero(i):
        for j in range(8):
            hist[i, pl.ds(j * LANES, LANES)] = zeros16

    lane = lax.iota(jnp.int32, LANES)

    rows_per_tile = HIST_ROWS // NSUB
    pltpu.sync_copy(hist.at[pl.ds(sid * rows_per_tile, rows_per_tile)],
                    shared.at[pl.ds(sid * rows_per_tile, rows_per_tile)])

    @plsc.parallel_loop(0, HIST_ROWS // LANES, unroll=2)
    def _iota(g):
        rowidx[pl.ds(g * LANES, LANES)] = g * LANES + lane

    pltpu.sync_copy(ei_hbm.at[:, pl.ds(base, EPT_MAX)], s_ei)
    pltpu.sync_copy(typ_hbm.at[pl.ds(base, EPT_MAX)], s_typ)

    ones16 = jnp.full((LANES,), 1.0, jnp.float32)

    @plsc.parallel_loop(0, GROUPS, unroll=4)
    def _scatter(i):
        off = i * LANES
        msk = lane < jnp.minimum(count - off, LANES)
        sr = s_ei[0, pl.ds(off, LANES)]
        ds_ = s_ei[1, pl.ds(off, LANES)]
        ty = s_typ[pl.ds(off, LANES)]
        row1 = ((ds_ >> 7) << 3) + ty
        plsc.addupdate_scatter(hist, [row1, ds_ & 127], ones16, mask=msk)
        row2 = ((sr >> 7) << 3) + (N_REL + ty)
        plsc.addupdate_scatter(hist, [row2, sr & 127], ones16,
                               mask=msk & (ds_ == 0))

    plsc.subcore_barrier()
    pltpu.sync_copy(hist, shared.at[rowidx], add=True)
    plsc.subcore_barrier()
    pltpu.sync_copy(
        shared.at[pl.ds(sid * rows_per_tile, rows_per_tile)],
        out_hbm.at[pl.ds(cid * HIST_ROWS + sid * rows_per_tile,
                         rows_per_tile)])


def _tc_body(h_ref, x_ref, cw_ref, w0_ref, gate_ref, mem_ref,
             wgt_ref, bg_ref, wst_ref, bs_ref, pg_ref, ps_ref):
    h = h_ref[...].reshape(NC, HIST_ROWS, 128)
    hsum = jnp.sum(h, axis=0)
    h3 = hsum.reshape(N_PAD // 128, 2 * N_REL, 128)
    dinv3 = lax.rsqrt(h3[:, 0:N_REL, :] + 1.0)
    c3 = h3[:, N_REL:2 * N_REL, :] * dinv3

    x = x_ref[...]
    xp = jnp.concatenate(
        [x, jnp.zeros((N_PAD - N_NODES, x.shape[1]), jnp.float32)], axis=0)
    x3 = xp.reshape(N_PAD // 128, 128, x.shape[1])
    sb = lax.dot_general(c3, x3, (((2,), (1,)), ((0,), (0,))),
                         preferred_element_type=jnp.float32)
    s = jnp.sum(sb, axis=0)

    dinv0 = dinv3[0, :, 0:1]
    x0 = x[0:1, :]
    y = dinv0 * s + (dinv0 * dinv0) * x0

    comp = jnp.zeros((1, x.shape[1]), jnp.float32)
    for r in range(N_REL):
        comp = comp + lax.dot_general(
            y[r:r + 1, :], cw_ref[r],
            (((1,), (1,)), ((), ())),
            preferred_element_type=jnp.float32)

    prev = jnp.dot(x0, w0_ref[...], preferred_element_type=jnp.float32)
    g = gate_ref[...]
    rg = g * (comp + prev) + (1.0 - g) * mem_ref[...]
    x1 = jnp.maximum(rg, 0.0)

    def head(wt_ref, b_ref, o_ref):
        logits = lax.dot_general(
            x1, wt_ref[...], (((1,), (1,)), ((), ())),
            preferred_element_type=jnp.float32) + b_ref[...]
        mx = jnp.max(logits, axis=1, keepdims=True)
        lse = jnp.log(jnp.sum(jnp.exp(logits - mx), axis=1,
                              keepdims=True)) + mx
        o_ref[...] = logits - lse

    head(wgt_ref, bg_ref, pg_ref)
    head(wst_ref, bs_ref, ps_ref)


def kernel(x, edge_index, edge_type, conv_W, W0, update_gate,
           lin2g_W, lin2g_b, lin2s_W, lin2s_b, memory_prev):
    hist = _get_sc_hist()(edge_index, edge_type)

    ng = lin2g_W.shape[0]
    ns = lin2s_W.shape[0]
    pg, ps = pl.pallas_call(
        _tc_body,
        out_shape=(
            jax.ShapeDtypeStruct((1, ng), jnp.float32),
            jax.ShapeDtypeStruct((1, ns), jnp.float32),
        ),
    )(hist, x, conv_W, W0,
      update_gate.reshape(1, 1), memory_prev.reshape(1, -1),
      lin2g_W, lin2g_b.reshape(1, -1),
      lin2s_W, lin2s_b.reshape(1, -1))
    return (pg, ps)
